# baseline (speedup 1.0000x reference)
import functools

import jax
import jax.numpy as jnp
from jax import lax
from jax.experimental import pallas as pl
from jax.experimental.pallas import tpu as pltpu
from jax.experimental.pallas import tpu_sc as plsc

N_NODES = 10000
N_EDGES = 320000
D = 128

NC = 2
NS = 16
NW = NC * NS
COLS = D // NW
L = 16

CH = 8000
N_CH = N_EDGES // CH
GROUPS = CH // L
TILE_W = COLS * N_NODES


@functools.partial(
    pl.kernel,
    out_type=jax.ShapeDtypeStruct((NW * TILE_W,), jnp.float32),
    mesh=plsc.VectorSubcoreMesh(core_axis_name="c", subcore_axis_name="s"),
    compiler_params=pltpu.CompilerParams(needs_layout_passes=False),
    scratch_types=[
        pltpu.VMEM((N_NODES,), jnp.float32),
        pltpu.VMEM((N_NODES,), jnp.float32),
        pltpu.VMEM((N_NODES,), jnp.float32),
        pltpu.VMEM((N_NODES,), jnp.float32),
        pltpu.VMEM((N_NODES,), jnp.float32),
        pltpu.VMEM((N_NODES,), jnp.float32),
        pltpu.VMEM((N_NODES,), jnp.float32),
        pltpu.VMEM((N_NODES,), jnp.float32),
        pltpu.VMEM((CH,), jnp.int32),
        pltpu.VMEM((CH,), jnp.int32),
        pltpu.VMEM((CH,), jnp.int32),
        pltpu.VMEM((CH,), jnp.int32),
        pltpu.SemaphoreType.DMA,
        pltpu.SemaphoreType.DMA,
        pltpu.SemaphoreType.DMA,
    ],
)
def _sc_segment_sum_t(xt_hbm, src_hbm, dst_hbm, out_hbm,
                      xc0, xc1, xc2, xc3, ac0, ac1, ac2, ac3,
                      src_a, dst_a, src_b, dst_b,
                      sem_x, sem_a, sem_b):
    c = lax.axis_index("c")
    s = lax.axis_index("s")
    t = s * NC + c
    xcs = (xc0, xc1, xc2, xc3)
    acs = (ac0, ac1, ac2, ac3)

    xcps = [
        pltpu.async_copy(
            xt_hbm.at[pl.ds(t * TILE_W + cc * N_NODES, N_NODES)], xcs[cc], sem_x)
        for cc in range(COLS)
    ]
    zero = jnp.zeros((L,), jnp.float32)

    def zbody(i, carry):
        for a in acs:
            a[pl.ds(i * L, L)] = zero
        return carry

    lax.fori_loop(0, N_NODES // L, zbody, 0)
    for cp in xcps:
        cp.wait()

    def fire(k, sbuf, dbuf, sem):
        pltpu.async_copy(src_hbm.at[pl.ds(k * CH, CH)], sbuf, sem)
        pltpu.async_copy(dst_hbm.at[pl.ds(k * CH, CH)], dbuf, sem)

    def drain(k, sbuf, dbuf, sem):
        pltpu.make_async_copy(src_hbm.at[pl.ds(k * CH, CH)], sbuf, sem).wait()
        pltpu.make_async_copy(dst_hbm.at[pl.ds(k * CH, CH)], dbuf, sem).wait()

    def process(sbuf, dbuf):
        def gbody(g, carry):
            s_vec = sbuf[pl.ds(g * L, L)]
            d_vec = dbuf[pl.ds(g * L, L)]
            for cc in range(COLS):
                v = plsc.load_gather(xcs[cc], [s_vec])
                plsc.addupdate_scatter(acs[cc], [d_vec], v)
            return carry

        lax.fori_loop(0, GROUPS, gbody, 0, unroll=8)

    fire(0, src_a, dst_a, sem_a)
    def mbody(m, carry):
        fire(2 * m + 1, src_b, dst_b, sem_b)
        drain(2 * m, src_a, dst_a, sem_a)
        process(src_a, dst_a)
        ka = jnp.minimum(2 * m + 2, N_CH - 1)
        fire(ka, src_a, dst_a, sem_a)
        drain(2 * m + 1, src_b, dst_b, sem_b)
        process(src_b, dst_b)
        return carry

    lax.fori_loop(0, N_CH // 2, mbody, 0)
    drain(N_CH - 1, src_a, dst_a, sem_a)

    for cc in range(COLS):
        pltpu.sync_copy(acs[cc], out_hbm.at[pl.ds(t * TILE_W + cc * N_NODES, N_NODES)])


def _finish_body(at_ref, w_ref, b_ref, o_ref):
    o_ref[...] = lax.dot_general(
        at_ref[...], w_ref[...],
        dimension_numbers=(((0,), (0,)), ((), ())),
        preferred_element_type=jnp.float32,
    ) + b_ref[...]


def _tc_finish(agg_t, weight, bias):
    return pl.pallas_call(
        _finish_body,
        out_shape=jax.ShapeDtypeStruct((N_NODES, D), jnp.float32),
    )(agg_t, weight, bias.reshape(1, D))


def kernel(x, edge_index, weight, bias):
    dst = edge_index[0]
    src = edge_index[1]
    xt_flat = x.T.reshape(NW * TILE_W)
    agg_flat = _sc_segment_sum_t(xt_flat, src, dst)
    agg_t = agg_flat.reshape(D, N_NODES)
    return _tc_finish(agg_t, weight, bias)


# speedup vs baseline: 2.1875x; 2.1875x over previous
import functools

import jax
import jax.numpy as jnp
from jax import lax
from jax.experimental import pallas as pl
from jax.experimental.pallas import tpu as pltpu
from jax.experimental.pallas import tpu_sc as plsc

N_NODES = 10000
N_EDGES = 320000
D = 128

NC = 2
NS = 16
NW = NC * NS
COLS = D // NW
L = 16

CH = 8000
N_CH = N_EDGES // CH
GROUPS = CH // L
TILE_W = COLS * N_NODES


@functools.partial(
    pl.kernel,
    out_type=jax.ShapeDtypeStruct((NW * TILE_W,), jnp.float32),
    mesh=plsc.VectorSubcoreMesh(core_axis_name="c", subcore_axis_name="s"),
    compiler_params=pltpu.CompilerParams(needs_layout_passes=False),
    scratch_types=[
        pltpu.VMEM((N_NODES,), jnp.float32),
        pltpu.VMEM((N_NODES,), jnp.float32),
        pltpu.VMEM((N_NODES,), jnp.float32),
        pltpu.VMEM((N_NODES,), jnp.float32),
        pltpu.VMEM((N_NODES,), jnp.float32),
        pltpu.VMEM((N_NODES,), jnp.float32),
        pltpu.VMEM((N_NODES,), jnp.float32),
        pltpu.VMEM((N_NODES,), jnp.float32),
        pltpu.VMEM((CH,), jnp.int32),
        pltpu.VMEM((CH,), jnp.int32),
        pltpu.VMEM((CH,), jnp.int32),
        pltpu.VMEM((CH,), jnp.int32),
        pltpu.SemaphoreType.DMA,
        pltpu.SemaphoreType.DMA,
        pltpu.SemaphoreType.DMA,
    ],
)
def _sc_segment_sum_t(xt_hbm, src_hbm, dst_hbm, out_hbm,
                      xc0, xc1, xc2, xc3, ac0, ac1, ac2, ac3,
                      src_a, dst_a, src_b, dst_b,
                      sem_x, sem_a, sem_b):
    c = lax.axis_index("c")
    s = lax.axis_index("s")
    t = s * NC + c
    xcs = (xc0, xc1, xc2, xc3)
    acs = (ac0, ac1, ac2, ac3)

    xcps = [
        pltpu.async_copy(
            xt_hbm.at[pl.ds(t * TILE_W + cc * N_NODES, N_NODES)], xcs[cc], sem_x)
        for cc in range(COLS)
    ]
    zero = jnp.zeros((L,), jnp.float32)

    def zbody(i, carry):
        for a in acs:
            a[pl.ds(i * L, L)] = zero
        return carry

    lax.fori_loop(0, N_NODES // L, zbody, 0)
    for cp in xcps:
        cp.wait()

    def fire(k, sbuf, dbuf, sem):
        pltpu.async_copy(src_hbm.at[pl.ds(k * CH, CH)], sbuf, sem)
        pltpu.async_copy(dst_hbm.at[pl.ds(k * CH, CH)], dbuf, sem)

    def drain(k, sbuf, dbuf, sem):
        pltpu.make_async_copy(src_hbm.at[pl.ds(k * CH, CH)], sbuf, sem).wait()
        pltpu.make_async_copy(dst_hbm.at[pl.ds(k * CH, CH)], dbuf, sem).wait()

    def process(sbuf, dbuf):
        @plsc.parallel_loop(0, GROUPS, unroll=8)
        def gbody(g):
            s_vec = sbuf[pl.ds(g * L, L)]
            d_vec = dbuf[pl.ds(g * L, L)]
            for cc in range(COLS):
                v = plsc.load_gather(xcs[cc], [s_vec])
                plsc.addupdate_scatter(acs[cc], [d_vec], v)

    fire(0, src_a, dst_a, sem_a)
    def mbody(m, carry):
        fire(2 * m + 1, src_b, dst_b, sem_b)
        drain(2 * m, src_a, dst_a, sem_a)
        process(src_a, dst_a)
        ka = jnp.minimum(2 * m + 2, N_CH - 1)
        fire(ka, src_a, dst_a, sem_a)
        drain(2 * m + 1, src_b, dst_b, sem_b)
        process(src_b, dst_b)
        return carry

    lax.fori_loop(0, N_CH // 2, mbody, 0)
    drain(N_CH - 1, src_a, dst_a, sem_a)

    for cc in range(COLS):
        pltpu.sync_copy(acs[cc], out_hbm.at[pl.ds(t * TILE_W + cc * N_NODES, N_NODES)])


def _finish_body(at_ref, w_ref, b_ref, o_ref):
    o_ref[...] = lax.dot_general(
        at_ref[...], w_ref[...],
        dimension_numbers=(((0,), (0,)), ((), ())),
        preferred_element_type=jnp.float32,
    ) + b_ref[...]


def _tc_finish(agg_t, weight, bias):
    return pl.pallas_call(
        _finish_body,
        out_shape=jax.ShapeDtypeStruct((N_NODES, D), jnp.float32),
    )(agg_t, weight, bias.reshape(1, D))


def kernel(x, edge_index, weight, bias):
    dst = edge_index[0]
    src = edge_index[1]
    xt_flat = x.T.reshape(NW * TILE_W)
    agg_flat = _sc_segment_sum_t(xt_flat, src, dst)
    agg_t = agg_flat.reshape(D, N_NODES)
    return _tc_finish(agg_t, weight, bias)
